# baseline (speedup 1.0000x reference)
import functools

import jax
import jax.numpy as jnp
from jax import lax
from jax.experimental import pallas as pl
from jax.experimental.pallas import tpu as pltpu
from jax.experimental.pallas import tpu_sc as plsc

N_NODES = 10000
N_EDGES = 320000
D = 128
EPS_LN = 1e-5

N_PAD = 10240
NC = 2
NS = 16
NW = NC * NS
CHUNK = 128
CPT = 80
NPH = 5
CPH = CPT // NPH
E_PAD = NW * CPT * CHUNK
ROWS_PT = N_PAD // NS
R_BLK = 1000

_MESH = plsc.VectorSubcoreMesh(
    core_axis_name="c", subcore_axis_name="s", num_cores=NC, num_subcores=NS
)

def _z16():
    return jnp.zeros((16,), jnp.float32)


@functools.partial(
    pl.kernel,
    out_type=jax.ShapeDtypeStruct((NC, N_PAD), jnp.float32),
    mesh=_MESH,
    scratch_types=[
        pltpu.VMEM((CPT, CHUNK), jnp.int32),
        pltpu.VMEM((CHUNK,), jnp.float32),
        pltpu.VMEM((ROWS_PT,), jnp.float32),
        pltpu.VMEM_SHARED((N_PAD,), jnp.float32),
        pltpu.SemaphoreType.DMA,
    ],
)
def _deg_kernel(ep_hbm, out_hbm, dstv, ones_v, zero_v, acc, sem):
    cid = lax.axis_index("c")
    sid = lax.axis_index("s")
    wid = cid * NS + sid
    pltpu.sync_copy(ep_hbm.at[1, pl.ds(wid * CPT, CPT)], dstv)
    for k in range(CHUNK // 16):
        ones_v[pl.ds(k * 16, 16)] = _z16() + 1.0

    def _zrow(r, carry):
        zero_v[pl.ds(r * 16, 16)] = _z16()
        return carry

    lax.fori_loop(0, ROWS_PT // 16, _zrow, 0)
    pltpu.sync_copy(zero_v, acc.at[pl.ds(sid * ROWS_PT, ROWS_PT)])
    plsc.subcore_barrier()

    for b in range(CPT // 16):
        def _fire(j, carry):
            pltpu.async_copy(ones_v, acc.at[dstv.at[b * 16 + j]], sem,
                             add=True)
            return carry

        lax.fori_loop(0, 16, _fire, 0)

        def _drain(j, carry):
            pltpu.make_async_copy(
                ones_v, acc.at[dstv.at[b * 16 + j]], sem).wait()
            return carry

        lax.fori_loop(0, 16, _drain, 0)
    plsc.subcore_barrier()
    pltpu.sync_copy(
        acc.at[pl.ds(sid * ROWS_PT, ROWS_PT)],
        out_hbm.at[cid, pl.ds(sid * ROWS_PT, ROWS_PT)],
    )


@functools.partial(
    pl.kernel,
    out_type=jax.ShapeDtypeStruct((NC, N_PAD, D), jnp.float32),
    mesh=_MESH,
    scratch_types=[
        pltpu.VMEM((CPH, CHUNK), jnp.int32),
        pltpu.VMEM((CPH, CHUNK), jnp.int32),
        pltpu.VMEM((CHUNK, D), jnp.float32),
        pltpu.VMEM((CHUNK, D), jnp.float32),
        pltpu.VMEM_SHARED((N_PAD, D), jnp.float32),
        pltpu.SemaphoreType.DMA,
        pltpu.SemaphoreType.DMA,
    ],
)
def _scat_kernel(h_hbm, ep_hbm, out_hbm, srcv, dstv, buf0, buf1,
                 acc, sem0, sem1):
    cid = lax.axis_index("c")
    sid = lax.axis_index("s")
    wid = cid * NS + sid

    def _zrow(r, carry):
        for c in range(D // 16):
            buf0[r, pl.ds(c * 16, 16)] = _z16()
        return carry

    lax.fori_loop(0, CHUNK, _zrow, 0)
    for k in range(ROWS_PT // CHUNK):
        pltpu.sync_copy(
            buf0, acc.at[pl.ds(sid * ROWS_PT + k * CHUNK, CHUNK)]
        )
    pltpu.sync_copy(ep_hbm.at[0, pl.ds(wid * CPT, CPH)], srcv)
    pltpu.sync_copy(ep_hbm.at[1, pl.ds(wid * CPT, CPH)], dstv)
    pltpu.async_copy(h_hbm.at[srcv.at[0]], buf0, sem0)
    pltpu.async_copy(h_hbm.at[srcv.at[1]], buf1, sem1)
    plsc.subcore_barrier()

    for ph in range(NPH):
        if ph > 0:
            base = wid * CPT + ph * CPH
            pltpu.sync_copy(ep_hbm.at[0, pl.ds(base, CPH)], srcv)
            pltpu.sync_copy(ep_hbm.at[1, pl.ds(base, CPH)], dstv)
            pltpu.async_copy(h_hbm.at[srcv.at[0]], buf0, sem0)
            pltpu.async_copy(h_hbm.at[srcv.at[1]], buf1, sem1)

        def _step(i, carry):
            j = 2 * i
            pltpu.make_async_copy(h_hbm.at[srcv.at[j]], buf0, sem0).wait()
            pltpu.sync_copy(buf0, acc.at[dstv.at[j]], add=True)

            @pl.when(j + 2 < CPH)
            def _():
                pltpu.async_copy(h_hbm.at[srcv.at[j + 2]], buf0, sem0)

            pltpu.make_async_copy(h_hbm.at[srcv.at[j + 1]], buf1, sem1).wait()
            pltpu.sync_copy(buf1, acc.at[dstv.at[j + 1]], add=True)

            @pl.when(j + 3 < CPH)
            def _():
                pltpu.async_copy(h_hbm.at[srcv.at[j + 3]], buf1, sem1)

            return carry

        lax.fori_loop(0, CPH // 2, _step, 0)

    plsc.subcore_barrier()
    pltpu.sync_copy(
        acc.at[pl.ds(sid * ROWS_PT, ROWS_PT)],
        out_hbm.at[cid, pl.ds(sid * ROWS_PT, ROWS_PT)],
    )


def _prep_body(degp_ref, x_ref, w_ref, hhat_ref, dis_ref):
    deg = degp_ref[:, 0:1] + degp_ref[:, 1:2] + 1.0
    dis = lax.rsqrt(deg)
    h = jnp.dot(x_ref[...], w_ref[...], preferred_element_type=jnp.float32)
    hhat_ref[...] = h * dis
    dis_ref[...] = dis


def _mid_body(s_ref, hhat_ref, dis_ref, b_ref, g_ref, be_ref, w_ref, out_ref):
    dis = dis_ref[...]
    t = (s_ref[0] + s_ref[1] + hhat_ref[...]) * dis + b_ref[...]
    mu = jnp.mean(t, axis=-1, keepdims=True)
    d = t - mu
    var = jnp.mean(d * d, axis=-1, keepdims=True)
    f = d * lax.rsqrt(var + EPS_LN) * g_ref[...] + be_ref[...]
    f = jnp.maximum(f, 0.0)
    out_ref[...] = (
        jnp.dot(f, w_ref[...], preferred_element_type=jnp.float32) * dis
    )


def _fin_body(s_ref, hhat_ref, dis_ref, b_ref, g_ref, be_ref, out_ref):
    t = (s_ref[0] + s_ref[1] + hhat_ref[...]) * dis_ref[...] + b_ref[...]
    mu = jnp.mean(t, axis=-1, keepdims=True)
    d = t - mu
    var = jnp.mean(d * d, axis=-1, keepdims=True)
    out_ref[...] = d * lax.rsqrt(var + EPS_LN) * g_ref[...] + be_ref[...]


def _row_spec(r):
    return pl.BlockSpec((r, D), lambda i: (i, 0))


def _col_spec(r):
    return pl.BlockSpec((r, 1), lambda i: (i, 0))


_FULL_W = pl.BlockSpec((D, D), lambda i: (0, 0))
_VEC = pl.BlockSpec((1, D), lambda i: (0, 0))


def _prep_call(degp, x, w0):
    return pl.pallas_call(
        _prep_body,
        grid=(N_NODES // R_BLK,),
        in_specs=[
            pl.BlockSpec((R_BLK, NC), lambda i: (i, 0)),
            _row_spec(R_BLK),
            _FULL_W,
        ],
        out_specs=[_row_spec(R_BLK), _col_spec(R_BLK)],
        out_shape=[
            jax.ShapeDtypeStruct((N_PAD, D), jnp.float32),
            jax.ShapeDtypeStruct((N_PAD, 1), jnp.float32),
        ],
    )(degp, x, w0)


def _mid_call(s, hhat, dis, b, g, be, w):
    return pl.pallas_call(
        _mid_body,
        grid=(N_NODES // R_BLK,),
        in_specs=[
            pl.BlockSpec((NC, R_BLK, D), lambda i: (0, i, 0)),
            _row_spec(R_BLK),
            _col_spec(R_BLK),
            _VEC, _VEC, _VEC,
            _FULL_W,
        ],
        out_specs=_row_spec(R_BLK),
        out_shape=jax.ShapeDtypeStruct((N_PAD, D), jnp.float32),
    )(s, hhat, dis, b.reshape(1, D), g.reshape(1, D), be.reshape(1, D), w)


def _fin_call(s, hhat, dis, b, g, be):
    return pl.pallas_call(
        _fin_body,
        grid=(N_NODES // R_BLK,),
        in_specs=[
            pl.BlockSpec((NC, R_BLK, D), lambda i: (0, i, 0)),
            _row_spec(R_BLK),
            _col_spec(R_BLK),
            _VEC, _VEC, _VEC,
        ],
        out_specs=_row_spec(R_BLK),
        out_shape=jax.ShapeDtypeStruct((N_NODES, D), jnp.float32),
    )(s, hhat, dis, b.reshape(1, D), g.reshape(1, D), be.reshape(1, D))


def kernel(x, edge_index, W0, b0, g0, be0, W1, b1, g1, be1, W2, b2, g2, be2):
    pad_ids = (
        jnp.arange(E_PAD - N_EDGES, dtype=jnp.int32) % (N_PAD - N_NODES)
    ) + N_NODES
    ep = jnp.concatenate(
        [edge_index, jnp.broadcast_to(pad_ids, (2, E_PAD - N_EDGES))], axis=1
    ).reshape(2, NW * CPT, CHUNK)

    degp = _deg_kernel(ep).T
    hhat0, dis = _prep_call(degp, x, W0)
    s0 = _scat_kernel(hhat0, ep)
    hhat1 = _mid_call(s0, hhat0, dis, b0, g0, be0, W1)
    s1 = _scat_kernel(hhat1, ep)
    hhat2 = _mid_call(s1, hhat1, dis, b1, g1, be1, W2)
    s2 = _scat_kernel(hhat2, ep)
    return _fin_call(s2, hhat2, dis, b2, g2, be2)


# speedup vs baseline: 1.0225x; 1.0225x over previous
import functools

import jax
import jax.numpy as jnp
from jax import lax
from jax.experimental import pallas as pl
from jax.experimental.pallas import tpu as pltpu
from jax.experimental.pallas import tpu_sc as plsc

N_NODES = 10000
N_EDGES = 320000
D = 128
EPS_LN = 1e-5

N_PAD = 10240
NC = 2
NS = 16
NW = NC * NS
CHUNK = 128
CPT = 80
NPH = 5
CPH = CPT // NPH
E_PAD = NW * CPT * CHUNK
ROWS_PT = N_PAD // NS
R_BLK = 2000

_MESH = plsc.VectorSubcoreMesh(
    core_axis_name="c", subcore_axis_name="s", num_cores=NC, num_subcores=NS
)

def _z16():
    return jnp.zeros((16,), jnp.float32)


@functools.partial(
    pl.kernel,
    out_type=jax.ShapeDtypeStruct((NC, N_PAD), jnp.float32),
    mesh=_MESH,
    scratch_types=[
        pltpu.VMEM((CPT, CHUNK), jnp.int32),
        pltpu.VMEM((CHUNK,), jnp.float32),
        pltpu.VMEM((ROWS_PT,), jnp.float32),
        pltpu.VMEM_SHARED((N_PAD,), jnp.float32),
        pltpu.SemaphoreType.DMA,
    ],
)
def _deg_kernel(ep_hbm, out_hbm, dstv, ones_v, zero_v, acc, sem):
    cid = lax.axis_index("c")
    sid = lax.axis_index("s")
    wid = cid * NS + sid
    pltpu.sync_copy(ep_hbm.at[1, pl.ds(wid * CPT, CPT)], dstv)
    for k in range(CHUNK // 16):
        ones_v[pl.ds(k * 16, 16)] = _z16() + 1.0

    def _zrow(r, carry):
        zero_v[pl.ds(r * 16, 16)] = _z16()
        return carry

    lax.fori_loop(0, ROWS_PT // 16, _zrow, 0)
    pltpu.sync_copy(zero_v, acc.at[pl.ds(sid * ROWS_PT, ROWS_PT)])
    plsc.subcore_barrier()

    for b in range(CPT // 16):
        def _fire(j, carry):
            pltpu.async_copy(ones_v, acc.at[dstv.at[b * 16 + j]], sem,
                             add=True)
            return carry

        lax.fori_loop(0, 16, _fire, 0)

        def _drain(j, carry):
            pltpu.make_async_copy(
                ones_v, acc.at[dstv.at[b * 16 + j]], sem).wait()
            return carry

        lax.fori_loop(0, 16, _drain, 0)
    plsc.subcore_barrier()
    pltpu.sync_copy(
        acc.at[pl.ds(sid * ROWS_PT, ROWS_PT)],
        out_hbm.at[cid, pl.ds(sid * ROWS_PT, ROWS_PT)],
    )


@functools.partial(
    pl.kernel,
    out_type=jax.ShapeDtypeStruct((NC, N_PAD, D), jnp.float32),
    mesh=_MESH,
    scratch_types=[
        pltpu.VMEM((CPH, CHUNK), jnp.int32),
        pltpu.VMEM((CPH, CHUNK), jnp.int32),
        pltpu.VMEM((CHUNK, D), jnp.float32),
        pltpu.VMEM((CHUNK, D), jnp.float32),
        pltpu.VMEM_SHARED((N_PAD, D), jnp.float32),
        pltpu.SemaphoreType.DMA,
        pltpu.SemaphoreType.DMA,
    ],
)
def _scat_kernel(h_hbm, ep_hbm, out_hbm, srcv, dstv, buf0, buf1,
                 acc, sem0, sem1):
    cid = lax.axis_index("c")
    sid = lax.axis_index("s")
    wid = cid * NS + sid

    def _zrow(r, carry):
        for c in range(D // 16):
            buf0[r, pl.ds(c * 16, 16)] = _z16()
        return carry

    lax.fori_loop(0, CHUNK, _zrow, 0)
    for k in range(ROWS_PT // CHUNK):
        pltpu.sync_copy(
            buf0, acc.at[pl.ds(sid * ROWS_PT + k * CHUNK, CHUNK)]
        )
    pltpu.sync_copy(ep_hbm.at[0, pl.ds(wid * CPT, CPH)], srcv)
    pltpu.sync_copy(ep_hbm.at[1, pl.ds(wid * CPT, CPH)], dstv)
    pltpu.async_copy(h_hbm.at[srcv.at[0]], buf0, sem0)
    pltpu.async_copy(h_hbm.at[srcv.at[1]], buf1, sem1)
    plsc.subcore_barrier()

    for ph in range(NPH):
        if ph > 0:
            base = wid * CPT + ph * CPH
            pltpu.sync_copy(ep_hbm.at[0, pl.ds(base, CPH)], srcv)
            pltpu.sync_copy(ep_hbm.at[1, pl.ds(base, CPH)], dstv)
            pltpu.async_copy(h_hbm.at[srcv.at[0]], buf0, sem0)
            pltpu.async_copy(h_hbm.at[srcv.at[1]], buf1, sem1)

        def _step(i, carry):
            j = 2 * i
            pltpu.make_async_copy(h_hbm.at[srcv.at[j]], buf0, sem0).wait()
            pltpu.sync_copy(buf0, acc.at[dstv.at[j]], add=True)

            @pl.when(j + 2 < CPH)
            def _():
                pltpu.async_copy(h_hbm.at[srcv.at[j + 2]], buf0, sem0)

            pltpu.make_async_copy(h_hbm.at[srcv.at[j + 1]], buf1, sem1).wait()
            pltpu.sync_copy(buf1, acc.at[dstv.at[j + 1]], add=True)

            @pl.when(j + 3 < CPH)
            def _():
                pltpu.async_copy(h_hbm.at[srcv.at[j + 3]], buf1, sem1)

            return carry

        lax.fori_loop(0, CPH // 2, _step, 0)

    plsc.subcore_barrier()
    pltpu.sync_copy(
        acc.at[pl.ds(sid * ROWS_PT, ROWS_PT)],
        out_hbm.at[cid, pl.ds(sid * ROWS_PT, ROWS_PT)],
    )


def _mm_body(x_ref, w_ref, h_ref):
    h_ref[...] = jnp.dot(
        x_ref[...], w_ref[...], preferred_element_type=jnp.float32
    )


def _scale_body(degp_ref, h_ref, hhat_ref, dis_ref):
    deg = degp_ref[:, 0:1] + degp_ref[:, 1:2] + 1.0
    dis = lax.rsqrt(deg)
    hhat_ref[...] = h_ref[...] * dis
    dis_ref[...] = dis


def _mid_body(s_ref, hhat_ref, dis_ref, b_ref, g_ref, be_ref, w_ref, out_ref):
    dis = dis_ref[...]
    t = (s_ref[0] + s_ref[1] + hhat_ref[...]) * dis + b_ref[...]
    mu = jnp.mean(t, axis=-1, keepdims=True)
    d = t - mu
    var = jnp.mean(d * d, axis=-1, keepdims=True)
    f = d * lax.rsqrt(var + EPS_LN) * g_ref[...] + be_ref[...]
    f = jnp.maximum(f, 0.0)
    out_ref[...] = (
        jnp.dot(f, w_ref[...], preferred_element_type=jnp.float32) * dis
    )


def _fin_body(s_ref, hhat_ref, dis_ref, b_ref, g_ref, be_ref, out_ref):
    t = (s_ref[0] + s_ref[1] + hhat_ref[...]) * dis_ref[...] + b_ref[...]
    mu = jnp.mean(t, axis=-1, keepdims=True)
    d = t - mu
    var = jnp.mean(d * d, axis=-1, keepdims=True)
    out_ref[...] = d * lax.rsqrt(var + EPS_LN) * g_ref[...] + be_ref[...]


def _row_spec(r):
    return pl.BlockSpec((r, D), lambda i: (i, 0))


def _col_spec(r):
    return pl.BlockSpec((r, 1), lambda i: (i, 0))


_FULL_W = pl.BlockSpec((D, D), lambda i: (0, 0))
_VEC = pl.BlockSpec((1, D), lambda i: (0, 0))


def _mm_call(x, w0):
    return pl.pallas_call(
        _mm_body,
        grid=(N_NODES // R_BLK,),
        in_specs=[_row_spec(R_BLK), _FULL_W],
        out_specs=_row_spec(R_BLK),
        out_shape=jax.ShapeDtypeStruct((N_NODES, D), jnp.float32),
    )(x, w0)


def _scale_call(degp, h0):
    return pl.pallas_call(
        _scale_body,
        grid=(N_NODES // R_BLK,),
        in_specs=[
            pl.BlockSpec((R_BLK, NC), lambda i: (i, 0)),
            _row_spec(R_BLK),
        ],
        out_specs=[_row_spec(R_BLK), _col_spec(R_BLK)],
        out_shape=[
            jax.ShapeDtypeStruct((N_PAD, D), jnp.float32),
            jax.ShapeDtypeStruct((N_PAD, 1), jnp.float32),
        ],
    )(degp, h0)


def _mid_call(s, hhat, dis, b, g, be, w):
    return pl.pallas_call(
        _mid_body,
        grid=(N_NODES // R_BLK,),
        in_specs=[
            pl.BlockSpec((NC, R_BLK, D), lambda i: (0, i, 0)),
            _row_spec(R_BLK),
            _col_spec(R_BLK),
            _VEC, _VEC, _VEC,
            _FULL_W,
        ],
        out_specs=_row_spec(R_BLK),
        out_shape=jax.ShapeDtypeStruct((N_PAD, D), jnp.float32),
    )(s, hhat, dis, b.reshape(1, D), g.reshape(1, D), be.reshape(1, D), w)


def _fin_call(s, hhat, dis, b, g, be):
    return pl.pallas_call(
        _fin_body,
        grid=(N_NODES // R_BLK,),
        in_specs=[
            pl.BlockSpec((NC, R_BLK, D), lambda i: (0, i, 0)),
            _row_spec(R_BLK),
            _col_spec(R_BLK),
            _VEC, _VEC, _VEC,
        ],
        out_specs=_row_spec(R_BLK),
        out_shape=jax.ShapeDtypeStruct((N_NODES, D), jnp.float32),
    )(s, hhat, dis, b.reshape(1, D), g.reshape(1, D), be.reshape(1, D))


def kernel(x, edge_index, W0, b0, g0, be0, W1, b1, g1, be1, W2, b2, g2, be2):
    pad_ids = (
        jnp.arange(E_PAD - N_EDGES, dtype=jnp.int32) % (N_PAD - N_NODES)
    ) + N_NODES
    ep = jnp.concatenate(
        [edge_index, jnp.broadcast_to(pad_ids, (2, E_PAD - N_EDGES))], axis=1
    ).reshape(2, NW * CPT, CHUNK)

    degp = _deg_kernel(ep).T
    h0 = _mm_call(x, W0)
    hhat0, dis = _scale_call(degp, h0)
    s0 = _scat_kernel(hhat0, ep)
    hhat1 = _mid_call(s0, hhat0, dis, b0, g0, be0, W1)
    s1 = _scat_kernel(hhat1, ep)
    hhat2 = _mid_call(s1, hhat1, dis, b1, g1, be1, W2)
    s2 = _scat_kernel(hhat2, ep)
    return _fin_call(s2, hhat2, dis, b2, g2, be2)


# speedup vs baseline: 1.0282x; 1.0056x over previous
import functools

import jax
import jax.numpy as jnp
from jax import lax
from jax.experimental import pallas as pl
from jax.experimental.pallas import tpu as pltpu
from jax.experimental.pallas import tpu_sc as plsc

N_NODES = 10000
N_EDGES = 320000
D = 128
EPS_LN = 1e-5

N_PAD = 10240
NC = 2
NS = 16
NW = NC * NS
CHUNK = 128
CPT = 80
NPH = 5
CPH = CPT // NPH
E_PAD = NW * CPT * CHUNK
ROWS_PT = N_PAD // NS
R_BLK = 2000

_MESH = plsc.VectorSubcoreMesh(
    core_axis_name="c", subcore_axis_name="s", num_cores=NC, num_subcores=NS
)

def _z16():
    return jnp.zeros((16,), jnp.float32)


@functools.partial(
    pl.kernel,
    out_type=jax.ShapeDtypeStruct((NC, N_PAD), jnp.float32),
    mesh=_MESH,
    scratch_types=[
        pltpu.VMEM((CPT, CHUNK), jnp.int32),
        pltpu.VMEM((CHUNK,), jnp.float32),
        pltpu.VMEM((ROWS_PT,), jnp.float32),
        pltpu.VMEM_SHARED((N_PAD,), jnp.float32),
        pltpu.SemaphoreType.DMA,
    ],
)
def _deg_kernel(ep_hbm, out_hbm, dstv, ones_v, zero_v, acc, sem):
    cid = lax.axis_index("c")
    sid = lax.axis_index("s")
    wid = cid * NS + sid
    pltpu.sync_copy(ep_hbm.at[1, pl.ds(wid * CPT, CPT)], dstv)
    for k in range(CHUNK // 16):
        ones_v[pl.ds(k * 16, 16)] = _z16() + 1.0

    def _zrow(r, carry):
        zero_v[pl.ds(r * 16, 16)] = _z16()
        return carry

    lax.fori_loop(0, ROWS_PT // 16, _zrow, 0)
    pltpu.sync_copy(zero_v, acc.at[pl.ds(sid * ROWS_PT, ROWS_PT)])
    plsc.subcore_barrier()

    for b in range(CPT // 16):
        def _fire(j, carry):
            pltpu.async_copy(ones_v, acc.at[dstv.at[b * 16 + j]], sem,
                             add=True)
            return carry

        lax.fori_loop(0, 16, _fire, 0)

        def _drain(j, carry):
            pltpu.make_async_copy(
                ones_v, acc.at[dstv.at[b * 16 + j]], sem).wait()
            return carry

        lax.fori_loop(0, 16, _drain, 0)
    plsc.subcore_barrier()
    pltpu.sync_copy(
        acc.at[pl.ds(sid * ROWS_PT, ROWS_PT)],
        out_hbm.at[cid, pl.ds(sid * ROWS_PT, ROWS_PT)],
    )


@functools.partial(
    pl.kernel,
    out_type=jax.ShapeDtypeStruct((NC, N_PAD, D), jnp.float32),
    mesh=_MESH,
    scratch_types=[
        pltpu.VMEM((CPH, CHUNK), jnp.int32),
        pltpu.VMEM((CPH, CHUNK), jnp.int32),
        pltpu.VMEM((CHUNK, D), jnp.float32),
        pltpu.VMEM((CHUNK, D), jnp.float32),
        pltpu.VMEM_SHARED((N_PAD, D), jnp.float32),
        pltpu.SemaphoreType.DMA,
        pltpu.SemaphoreType.DMA,
    ],
)
def _scat_kernel(h_hbm, ep_hbm, out_hbm, srcv, dstv, buf0, buf1,
                 acc, sem0, sem1):
    cid = lax.axis_index("c")
    sid = lax.axis_index("s")
    wid = cid * NS + sid

    def _zrow(r, carry):
        for c in range(D // 16):
            buf0[r, pl.ds(c * 16, 16)] = _z16()
        return carry

    lax.fori_loop(0, CHUNK, _zrow, 0)
    for k in range(ROWS_PT // CHUNK):
        pltpu.sync_copy(
            buf0, acc.at[pl.ds(sid * ROWS_PT + k * CHUNK, CHUNK)]
        )
    pltpu.sync_copy(ep_hbm.at[0, pl.ds(wid * CPT, CPH)], srcv)
    pltpu.sync_copy(ep_hbm.at[1, pl.ds(wid * CPT, CPH)], dstv)
    pltpu.async_copy(h_hbm.at[srcv.at[0]], buf0, sem0)
    pltpu.async_copy(h_hbm.at[srcv.at[1]], buf1, sem1)
    plsc.subcore_barrier()

    for ph in range(NPH):
        if ph > 0:
            base = wid * CPT + ph * CPH
            pltpu.sync_copy(ep_hbm.at[0, pl.ds(base, CPH)], srcv)
            pltpu.sync_copy(ep_hbm.at[1, pl.ds(base, CPH)], dstv)
            pltpu.async_copy(h_hbm.at[srcv.at[0]], buf0, sem0)
            pltpu.async_copy(h_hbm.at[srcv.at[1]], buf1, sem1)

        def _step(i, carry):
            j = 2 * i
            pltpu.make_async_copy(h_hbm.at[srcv.at[j]], buf0, sem0).wait()
            pltpu.sync_copy(buf0, acc.at[dstv.at[j]], add=True)

            @pl.when(j + 2 < CPH)
            def _():
                pltpu.async_copy(h_hbm.at[srcv.at[j + 2]], buf0, sem0)

            pltpu.make_async_copy(h_hbm.at[srcv.at[j + 1]], buf1, sem1).wait()
            pltpu.sync_copy(buf1, acc.at[dstv.at[j + 1]], add=True)

            @pl.when(j + 3 < CPH)
            def _():
                pltpu.async_copy(h_hbm.at[srcv.at[j + 3]], buf1, sem1)

            return carry

        lax.fori_loop(0, CPH // 2, _step, 0)

    plsc.subcore_barrier()
    pltpu.sync_copy(
        acc.at[pl.ds(sid * ROWS_PT, ROWS_PT)],
        out_hbm.at[cid, pl.ds(sid * ROWS_PT, ROWS_PT)],
    )


def _prep_body(degp_ref, x_ref, w_ref, hhat_ref, dis_ref):
    deg = degp_ref[:, 0:1] + degp_ref[:, 1:2] + 1.0
    dis = lax.rsqrt(deg)
    h = jnp.dot(x_ref[...], w_ref[...], preferred_element_type=jnp.float32)
    hhat_ref[...] = h * dis
    dis_ref[...] = dis


def _mid_body(s_ref, hhat_ref, dis_ref, b_ref, g_ref, be_ref, w_ref, out_ref):
    dis = dis_ref[...]
    t = (s_ref[0] + s_ref[1] + hhat_ref[...]) * dis + b_ref[...]
    mu = jnp.mean(t, axis=-1, keepdims=True)
    d = t - mu
    var = jnp.mean(d * d, axis=-1, keepdims=True)
    f = d * lax.rsqrt(var + EPS_LN) * g_ref[...] + be_ref[...]
    f = jnp.maximum(f, 0.0)
    out_ref[...] = (
        jnp.dot(f, w_ref[...], preferred_element_type=jnp.float32) * dis
    )


def _fin_body(s_ref, hhat_ref, dis_ref, b_ref, g_ref, be_ref, out_ref):
    t = (s_ref[0] + s_ref[1] + hhat_ref[...]) * dis_ref[...] + b_ref[...]
    mu = jnp.mean(t, axis=-1, keepdims=True)
    d = t - mu
    var = jnp.mean(d * d, axis=-1, keepdims=True)
    out_ref[...] = d * lax.rsqrt(var + EPS_LN) * g_ref[...] + be_ref[...]


def _row_spec(r):
    return pl.BlockSpec((r, D), lambda i: (i, 0))


def _col_spec(r):
    return pl.BlockSpec((r, 1), lambda i: (i, 0))


_FULL_W = pl.BlockSpec((D, D), lambda i: (0, 0))
_VEC = pl.BlockSpec((1, D), lambda i: (0, 0))


def _prep_call(degp, x, w0):
    return pl.pallas_call(
        _prep_body,
        grid=(N_NODES // R_BLK,),
        in_specs=[
            pl.BlockSpec((R_BLK, NC), lambda i: (i, 0)),
            _row_spec(R_BLK),
            _FULL_W,
        ],
        out_specs=[_row_spec(R_BLK), _col_spec(R_BLK)],
        out_shape=[
            jax.ShapeDtypeStruct((N_PAD, D), jnp.float32),
            jax.ShapeDtypeStruct((N_PAD, 1), jnp.float32),
        ],
    )(degp, x, w0)


def _mid_call(s, hhat, dis, b, g, be, w):
    return pl.pallas_call(
        _mid_body,
        grid=(N_NODES // R_BLK,),
        in_specs=[
            pl.BlockSpec((NC, R_BLK, D), lambda i: (0, i, 0)),
            _row_spec(R_BLK),
            _col_spec(R_BLK),
            _VEC, _VEC, _VEC,
            _FULL_W,
        ],
        out_specs=_row_spec(R_BLK),
        out_shape=jax.ShapeDtypeStruct((N_PAD, D), jnp.float32),
    )(s, hhat, dis, b.reshape(1, D), g.reshape(1, D), be.reshape(1, D), w)


def _fin_call(s, hhat, dis, b, g, be):
    return pl.pallas_call(
        _fin_body,
        grid=(N_NODES // R_BLK,),
        in_specs=[
            pl.BlockSpec((NC, R_BLK, D), lambda i: (0, i, 0)),
            _row_spec(R_BLK),
            _col_spec(R_BLK),
            _VEC, _VEC, _VEC,
        ],
        out_specs=_row_spec(R_BLK),
        out_shape=jax.ShapeDtypeStruct((N_NODES, D), jnp.float32),
    )(s, hhat, dis, b.reshape(1, D), g.reshape(1, D), be.reshape(1, D))


def kernel(x, edge_index, W0, b0, g0, be0, W1, b1, g1, be1, W2, b2, g2, be2):
    pad_ids = (
        jnp.arange(E_PAD - N_EDGES, dtype=jnp.int32) % (N_PAD - N_NODES)
    ) + N_NODES
    ep = jnp.concatenate(
        [edge_index, jnp.broadcast_to(pad_ids, (2, E_PAD - N_EDGES))], axis=1
    ).reshape(2, NW * CPT, CHUNK)

    degp = _deg_kernel(ep).T
    hhat0, dis = _prep_call(degp, x, W0)
    s0 = _scat_kernel(hhat0, ep)
    hhat1 = _mid_call(s0, hhat0, dis, b0, g0, be0, W1)
    s1 = _scat_kernel(hhat1, ep)
    hhat2 = _mid_call(s1, hhat1, dis, b1, g1, be1, W2)
    s2 = _scat_kernel(hhat2, ep)
    return _fin_call(s2, hhat2, dis, b2, g2, be2)
